# baseline (speedup 1.0000x reference)
import functools

import jax
import jax.numpy as jnp
from jax.experimental import pallas as pl
from jax.experimental.pallas import tpu as pltpu

_K = 20
_TEMP = 0.01


def _block_kernel(feat_ref, sfeat_ref, x_ref, out_ref, w_ref):
    f = feat_ref[0]
    s = sfeat_ref[0]
    xb = x_ref[0]

    fn = jnp.sum(f * f, axis=1, keepdims=True)
    sn = jnp.sum(s * s, axis=1, keepdims=True).T
    dot = jax.lax.dot_general(
        f, s, (((1,), (1,)), ((), ())),
        preferred_element_type=jnp.float32)
    ds = jnp.maximum(fn + sn - 2.0 * dot, 0.0)
    w = jnp.exp(-_TEMP * ds)

    bn, m = w.shape

    w_ref[...] = w

    def chunk_body(c, carry):
        wc = w_ref[pl.ds(c * 8, 8), :]
        wm = wc
        removed = jnp.zeros((8, 1), jnp.float32)
        thr = jnp.zeros((8, 1), jnp.float32)
        for j in range(_K):
            mx = jnp.max(wm, axis=1, keepdims=True)
            thr = jnp.where(removed < _K, mx, thr)
            if j < _K - 1:
                eq = wm == mx
                removed = removed + jnp.sum(
                    jnp.where(eq, 1.0, 0.0), axis=1, keepdims=True)
                wm = jnp.where(eq, -jnp.inf, wm)
        ac = jnp.where(wc >= thr, wc, 0.0)
        norm = jnp.sqrt(jnp.sum(ac * ac, axis=1, keepdims=True))
        w_ref[pl.ds(c * 8, 8), :] = ac / jnp.maximum(norm, 1e-12)
        return carry

    jax.lax.fori_loop(0, bn // 8, chunk_body, 0)
    att = w_ref[...]

    out_ref[0] = jax.lax.dot_general(
        att, xb, (((1,), (0,)), ((), ())),
        preferred_element_type=jnp.float32)


@functools.partial(jax.jit, static_argnames=("bn",))
def _run(x, feat, sfeat, bn):
    b, n, c = feat.shape
    _, m, _ = x.shape
    grid = (b, n // bn)
    return pl.pallas_call(
        _block_kernel,
        grid=grid,
        in_specs=[
            pl.BlockSpec((1, bn, c), lambda bi, ni: (bi, ni, 0)),
            pl.BlockSpec((1, m, c), lambda bi, ni: (bi, 0, 0)),
            pl.BlockSpec((1, m, c), lambda bi, ni: (bi, 0, 0)),
        ],
        out_specs=pl.BlockSpec((1, bn, c), lambda bi, ni: (bi, ni, 0)),
        out_shape=jax.ShapeDtypeStruct((b, n, c), jnp.float32),
        scratch_shapes=[pltpu.VMEM((bn, m), jnp.float32)],
    )(feat, sfeat, x)


def kernel(x, feat_before_pooling, feat_after_pooling):
    n = feat_before_pooling.shape[1]
    bn = 512 if n % 512 == 0 else n
    return _run(x, feat_before_pooling, feat_after_pooling, bn)


# speedup vs baseline: 3.5747x; 3.5747x over previous
import functools

import jax
import jax.numpy as jnp
from jax.experimental import pallas as pl
from jax.experimental.pallas import tpu as pltpu

_K = 20
_TEMP = 0.01


def _block_kernel(feat_ref, sfeat_ref, x_ref, out_ref, w_ref):
    f = feat_ref[0]
    s = sfeat_ref[0]
    xb = x_ref[0]

    fn = jnp.sum(f * f, axis=1, keepdims=True)
    sn = jnp.sum(s * s, axis=1, keepdims=True).T
    dot = jax.lax.dot_general(
        f, s, (((1,), (1,)), ((), ())),
        preferred_element_type=jnp.float32)
    ds = jnp.maximum(fn + sn - 2.0 * dot, 0.0)
    w = jnp.exp(-_TEMP * ds)

    bn, m = w.shape

    del w_ref
    w3 = w.reshape(bn, 8, m // 8) if m % 8 == 0 else w.reshape(bn, 1, m)
    wm = w3
    removed = jnp.zeros((bn, 1, 1), jnp.float32)
    thr = jnp.zeros((bn, 1, 1), jnp.float32)
    for j in range(_K):
        mx = jnp.max(wm, axis=(1, 2), keepdims=True)
        thr = jnp.where(removed < _K, mx, thr)
        if j < _K - 1:
            eq = wm == mx
            removed = removed + jnp.sum(
                jnp.where(eq, 1.0, 0.0), axis=(1, 2), keepdims=True)
            wm = jnp.where(eq, -jnp.inf, wm)

    att = jnp.where(w3 >= thr, w3, 0.0)
    norm = jnp.sqrt(jnp.sum(att * att, axis=(1, 2), keepdims=True))
    att = (att / jnp.maximum(norm, 1e-12)).reshape(bn, m)

    out_ref[0] = jax.lax.dot_general(
        att, xb, (((1,), (0,)), ((), ())),
        preferred_element_type=jnp.float32)


@functools.partial(jax.jit, static_argnames=("bn",))
def _run(x, feat, sfeat, bn):
    b, n, c = feat.shape
    _, m, _ = x.shape
    grid = (b, n // bn)
    return pl.pallas_call(
        _block_kernel,
        grid=grid,
        in_specs=[
            pl.BlockSpec((1, bn, c), lambda bi, ni: (bi, ni, 0)),
            pl.BlockSpec((1, m, c), lambda bi, ni: (bi, 0, 0)),
            pl.BlockSpec((1, m, c), lambda bi, ni: (bi, 0, 0)),
        ],
        out_specs=pl.BlockSpec((1, bn, c), lambda bi, ni: (bi, ni, 0)),
        out_shape=jax.ShapeDtypeStruct((b, n, c), jnp.float32),
        scratch_shapes=[pltpu.VMEM((bn, m), jnp.float32)],
    )(feat, sfeat, x)


def kernel(x, feat_before_pooling, feat_after_pooling):
    n = feat_before_pooling.shape[1]
    bn = 512 if n % 512 == 0 else n
    return _run(x, feat_before_pooling, feat_after_pooling, bn)


# speedup vs baseline: 9.4815x; 2.6524x over previous
import functools

import jax
import jax.numpy as jnp
from jax.experimental import pallas as pl
from jax.experimental.pallas import tpu as pltpu

_K = 20
_TEMP = 0.01


def _block_kernel(feat_ref, sfeat_ref, x_ref, out_ref, w_ref):
    f = feat_ref[0]
    s = sfeat_ref[0]
    xb = x_ref[0]

    fn = jnp.sum(f * f, axis=1, keepdims=True)
    sn = jnp.sum(s * s, axis=1, keepdims=True).T
    dot = jax.lax.dot_general(
        f, s, (((1,), (1,)), ((), ())),
        preferred_element_type=jnp.float32)
    ds = jnp.maximum(fn + sn - 2.0 * dot, 0.0)
    w = jnp.exp(-_TEMP * ds)

    bn, m = w.shape

    del w_ref
    wm = w
    ones_cnt = jnp.ones((m, 128), jnp.float32)
    removed = jnp.zeros((bn, 1), jnp.float32)
    thr = jnp.zeros((bn, 1), jnp.float32)
    for j in range(_K):
        mx = jnp.max(wm, axis=1, keepdims=True)
        thr = jnp.where(removed < _K, mx, thr)
        if j < _K - 1:
            eq = wm == mx
            ind = jnp.where(eq, 1.0, 0.0)
            cnt = jax.lax.dot_general(
                ind, ones_cnt, (((1,), (0,)), ((), ())),
                preferred_element_type=jnp.float32)[:, :1]
            removed = removed + cnt
            wm = jnp.where(eq, -jnp.inf, wm)

    att = jnp.where(w >= thr, w, 0.0)
    norm = jnp.sqrt(jnp.sum(att * att, axis=1, keepdims=True))
    att = att / jnp.maximum(norm, 1e-12)

    out_ref[0] = jax.lax.dot_general(
        att, xb, (((1,), (0,)), ((), ())),
        preferred_element_type=jnp.float32)


@functools.partial(jax.jit, static_argnames=("bn",))
def _run(x, feat, sfeat, bn):
    b, n, c = feat.shape
    _, m, _ = x.shape
    grid = (b, n // bn)
    return pl.pallas_call(
        _block_kernel,
        grid=grid,
        in_specs=[
            pl.BlockSpec((1, bn, c), lambda bi, ni: (bi, ni, 0)),
            pl.BlockSpec((1, m, c), lambda bi, ni: (bi, 0, 0)),
            pl.BlockSpec((1, m, c), lambda bi, ni: (bi, 0, 0)),
        ],
        out_specs=pl.BlockSpec((1, bn, c), lambda bi, ni: (bi, ni, 0)),
        out_shape=jax.ShapeDtypeStruct((b, n, c), jnp.float32),
        scratch_shapes=[pltpu.VMEM((bn, m), jnp.float32)],
    )(feat, sfeat, x)


def kernel(x, feat_before_pooling, feat_after_pooling):
    n = feat_before_pooling.shape[1]
    bn = 512 if n % 512 == 0 else n
    return _run(x, feat_before_pooling, feat_after_pooling, bn)


# speedup vs baseline: 10.2987x; 1.0862x over previous
import functools

import jax
import jax.numpy as jnp
from jax.experimental import pallas as pl
from jax.experimental.pallas import tpu as pltpu

_K = 20
_TEMP = 0.01


def _block_kernel(feat_ref, sfeat_ref, x_ref, out_ref):
    f = feat_ref[0]
    s = sfeat_ref[0]
    xb = x_ref[0]

    fn = jnp.sum(f * f, axis=1, keepdims=True)
    sn = jnp.sum(s * s, axis=1, keepdims=True).T
    dot = jax.lax.dot_general(
        f, s, (((1,), (1,)), ((), ())),
        preferred_element_type=jnp.float32)
    ds = jnp.maximum(fn + sn - 2.0 * dot, 0.0)
    w = jnp.exp(-_TEMP * ds)

    bn, m = w.shape

    wm = w
    removed = jnp.zeros((bn, 1), jnp.float32)
    thr = jnp.zeros((bn, 1), jnp.float32)
    for j in range(_K):
        mx = jnp.max(wm, axis=1, keepdims=True)
        thr = jnp.where(removed < _K, mx, thr)
        if j < _K - 1:
            eq = wm == mx
            removed = removed + jnp.sum(
                jnp.where(eq, 1.0, 0.0), axis=1, keepdims=True)
            wm = jnp.where(eq, -jnp.inf, wm)

    att = jnp.where(w >= thr, w, 0.0)
    norm = jnp.sqrt(jnp.sum(att * att, axis=1, keepdims=True))
    att = att / jnp.maximum(norm, 1e-12)

    out_ref[0] = jax.lax.dot_general(
        att, xb, (((1,), (0,)), ((), ())),
        preferred_element_type=jnp.float32)


@functools.partial(jax.jit, static_argnames=("bn",))
def _run(x, feat, sfeat, bn):
    b, n, c = feat.shape
    _, m, _ = x.shape
    grid = (b, n // bn)
    return pl.pallas_call(
        _block_kernel,
        grid=grid,
        in_specs=[
            pl.BlockSpec((1, bn, c), lambda bi, ni: (bi, ni, 0)),
            pl.BlockSpec((1, m, c), lambda bi, ni: (bi, 0, 0)),
            pl.BlockSpec((1, m, c), lambda bi, ni: (bi, 0, 0)),
        ],
        out_specs=pl.BlockSpec((1, bn, c), lambda bi, ni: (bi, ni, 0)),
        out_shape=jax.ShapeDtypeStruct((b, n, c), jnp.float32),
    )(feat, sfeat, x)


def kernel(x, feat_before_pooling, feat_after_pooling):
    n = feat_before_pooling.shape[1]
    bn = 256 if n % 256 == 0 else n
    return _run(x, feat_before_pooling, feat_after_pooling, bn)
